# Initial kernel scaffold; baseline (speedup 1.0000x reference)
import functools

import jax
import jax.numpy as jnp
from jax import lax
from jax.experimental import pallas as pl
from jax.experimental.pallas import tpu as pltpu
from jax.experimental.pallas import tpu_sc as plsc

F32 = jnp.float32


def _edge_proj_body(ea_ref, we_ref, be_ref, out_ref):
  out_ref[...] = (
      jnp.dot(ea_ref[...], we_ref[...], preferred_element_type=F32)
      + be_ref[...]
  )


def _edge_proj(edge_attr, We_l, be_l):
  E, K = edge_attr.shape
  H = We_l.shape[1]
  BE = 4000
  grid = E // BE
  return pl.pallas_call(
      _edge_proj_body,
      grid=(grid,),
      in_specs=[
          pl.BlockSpec((BE, K), lambda i: (i, 0)),
          pl.BlockSpec((K, H), lambda i: (0, 0)),
          pl.BlockSpec((1, H), lambda i: (0, 0)),
      ],
      out_specs=pl.BlockSpec((BE, H), lambda i: (i, 0)),
      out_shape=jax.ShapeDtypeStruct((E, H), F32),
  )(edge_attr, We_l, be_l.reshape(1, H))


def _node_update_body(relu_out, h_ref, a0_ref, a1_ref, w1_ref, b1_ref,
                      w2_ref, b2_ref, out_ref):
  z = h_ref[...] + a0_ref[...] + a1_ref[...]
  z = jnp.maximum(
      jnp.dot(z, w1_ref[...], preferred_element_type=F32) + b1_ref[...], 0.0)
  z = jnp.dot(z, w2_ref[...], preferred_element_type=F32) + b2_ref[...]
  if relu_out:
    z = jnp.maximum(z, 0.0)
  out_ref[...] = z


def _node_update(h, a0, a1, W1_l, b1_l, W2_l, b2_l, relu_out):
  N, H = h.shape
  BN = 2000
  grid = N // BN
  row_spec = pl.BlockSpec((BN, H), lambda i: (i, 0))
  mat_spec = pl.BlockSpec((H, H), lambda i: (0, 0))
  vec_spec = pl.BlockSpec((1, H), lambda i: (0, 0))
  return pl.pallas_call(
      functools.partial(_node_update_body, relu_out),
      grid=(grid,),
      in_specs=[row_spec, row_spec, row_spec, mat_spec, vec_spec,
                mat_spec, vec_spec],
      out_specs=row_spec,
      out_shape=jax.ShapeDtypeStruct((N, H), F32),
  )(h, a0, a1, W1_l, b1_l.reshape(1, H), W2_l, b2_l.reshape(1, H))


_NC, _NS = 2, 16
_NW = _NC * _NS
_CHUNK = 128


def _make_sc_edge_pass(N, E, H):
  EW = E // _NW
  n_full = EW // _CHUNK
  rem = EW - n_full * _CHUNK
  rows_per_tile = N // _NS

  def body(h_hbm, e_hbm, src_hbm, dst_hbm, zeros_hbm, out_hbm,
           acc_sh, src_v, dst_v, rows_v, ev_v,
           src_r, dst_r, rows_r, ev_r, sem):
    c = lax.axis_index("c")
    s = lax.axis_index("s")
    wid = c * _NS + s

    pltpu.sync_copy(zeros_hbm.at[pl.ds(s * rows_per_tile, rows_per_tile)],
                    acc_sh.at[pl.ds(s * rows_per_tile, rows_per_tile)])
    plsc.subcore_barrier()

    def do_chunk(base, src_i, dst_i, rows_i, ev_i, nb):
      pltpu.sync_copy(src_hbm.at[pl.ds(base, nb)], src_i)
      pltpu.sync_copy(dst_hbm.at[pl.ds(base, nb)], dst_i)
      pltpu.async_copy(h_hbm.at[src_i], rows_i, sem).wait()
      pltpu.sync_copy(e_hbm.at[pl.ds(base, nb)], ev_i)

      def row(b, carry):
        for j in range(H // 16):
          sl = pl.ds(j * 16, 16)
          v = rows_i[b, sl] + ev_i[b, sl]
          rows_i[b, sl] = jnp.maximum(v, 0.0)
        return carry
      lax.fori_loop(0, nb, row, 0)

      pltpu.sync_copy(rows_i, acc_sh.at[dst_i], add=True)

    def chunk(i, carry):
      do_chunk(wid * EW + i * _CHUNK, src_v, dst_v, rows_v, ev_v, _CHUNK)
      return carry
    lax.fori_loop(0, n_full, chunk, 0)
    if rem:
      do_chunk(wid * EW + n_full * _CHUNK, src_r, dst_r, rows_r, ev_r, rem)

    plsc.subcore_barrier()
    pltpu.sync_copy(acc_sh.at[pl.ds(s * rows_per_tile, rows_per_tile)],
                    out_hbm.at[c, pl.ds(s * rows_per_tile, rows_per_tile)])

  mesh = plsc.VectorSubcoreMesh(core_axis_name="c", subcore_axis_name="s")
  scratch = [
      pltpu.VMEM_SHARED((N, H), F32),
      pltpu.VMEM((_CHUNK,), jnp.int32),
      pltpu.VMEM((_CHUNK,), jnp.int32),
      pltpu.VMEM((_CHUNK, H), F32),
      pltpu.VMEM((_CHUNK, H), F32),
      pltpu.VMEM((max(rem, 8),), jnp.int32),
      pltpu.VMEM((max(rem, 8),), jnp.int32),
      pltpu.VMEM((max(rem, 8), H), F32),
      pltpu.VMEM((max(rem, 8), H), F32),
      pltpu.SemaphoreType.DMA,
  ]
  return pl.kernel(
      body,
      out_type=jax.ShapeDtypeStruct((_NC, N, H), F32),
      mesh=mesh,
      scratch_types=scratch,
  )


def kernel(x, edge_index, edge_attr, We, be, W1, b1, W2, b2):
  N, H = x.shape[0], We.shape[2]
  E = edge_attr.shape[0]
  src = edge_index[0].astype(jnp.int32)
  dst = edge_index[1].astype(jnp.int32)
  zeros = jnp.zeros((N, H), dtype=F32)
  sc_edge_pass = _make_sc_edge_pass(N, E, H)

  num_layers = We.shape[0]
  h = x
  for l in range(num_layers):
    e = _edge_proj(edge_attr, We[l], be[l])
    agg = sc_edge_pass(h, e, src, dst, zeros)
    h = _node_update(h, agg[0], agg[1], W1[l], b1[l], W2[l], b2[l],
                     relu_out=(l < num_layers - 1))
  return h


# speedup vs baseline: 2.9202x; 2.9202x over previous
import functools

import jax
import jax.numpy as jnp
from jax import lax
from jax.experimental import pallas as pl
from jax.experimental.pallas import tpu as pltpu
from jax.experimental.pallas import tpu_sc as plsc

F32 = jnp.float32


def _edge_proj_body(ea_ref, we_ref, be_ref, out_ref):
  out_ref[...] = (
      jnp.dot(ea_ref[...], we_ref[...], preferred_element_type=F32)
      + be_ref[...]
  )


def _edge_proj(edge_attr, We_l, be_l):
  E, K = edge_attr.shape
  H = We_l.shape[1]
  BE = 4000
  grid = E // BE
  return pl.pallas_call(
      _edge_proj_body,
      grid=(grid,),
      in_specs=[
          pl.BlockSpec((BE, K), lambda i: (i, 0)),
          pl.BlockSpec((K, H), lambda i: (0, 0)),
          pl.BlockSpec((1, H), lambda i: (0, 0)),
      ],
      out_specs=pl.BlockSpec((BE, H), lambda i: (i, 0)),
      out_shape=jax.ShapeDtypeStruct((E, H), F32),
  )(edge_attr, We_l, be_l.reshape(1, H))


def _node_update_body(relu_out, h_ref, a0_ref, a1_ref, w1_ref, b1_ref,
                      w2_ref, b2_ref, out_ref):
  z = h_ref[...] + a0_ref[...] + a1_ref[...]
  z = jnp.maximum(
      jnp.dot(z, w1_ref[...], preferred_element_type=F32) + b1_ref[...], 0.0)
  z = jnp.dot(z, w2_ref[...], preferred_element_type=F32) + b2_ref[...]
  if relu_out:
    z = jnp.maximum(z, 0.0)
  out_ref[...] = z


def _node_update(h, a0, a1, W1_l, b1_l, W2_l, b2_l, relu_out):
  N, H = h.shape
  BN = 2000
  grid = N // BN
  row_spec = pl.BlockSpec((BN, H), lambda i: (i, 0))
  mat_spec = pl.BlockSpec((H, H), lambda i: (0, 0))
  vec_spec = pl.BlockSpec((1, H), lambda i: (0, 0))
  return pl.pallas_call(
      functools.partial(_node_update_body, relu_out),
      grid=(grid,),
      in_specs=[row_spec, row_spec, row_spec, mat_spec, vec_spec,
                mat_spec, vec_spec],
      out_specs=row_spec,
      out_shape=jax.ShapeDtypeStruct((N, H), F32),
  )(h, a0, a1, W1_l, b1_l.reshape(1, H), W2_l, b2_l.reshape(1, H))


_NC, _NS = 2, 16
_NW = _NC * _NS
_CHUNK = 128


def _make_sc_edge_pass(N, E, H):
  EW = E // _NW
  n_full = EW // _CHUNK
  rem = EW - n_full * _CHUNK
  rows_per_tile = ((N + _NS - 1) // _NS + 7) // 8 * 8
  N_pad = rows_per_tile * _NS

  def body(h_hbm, e_hbm, src_hbm, dst_hbm, zeros_hbm, out_hbm,
           acc_sh, src_v, dst_v, rows_v, ev_v,
           src_r, dst_r, rows_r, ev_r, sem):
    c = lax.axis_index("c")
    s = lax.axis_index("s")
    wid = c * _NS + s

    pltpu.sync_copy(zeros_hbm.at[pl.ds(s * rows_per_tile, rows_per_tile)],
                    acc_sh.at[pl.ds(s * rows_per_tile, rows_per_tile)])
    plsc.subcore_barrier()

    def do_chunk(base, src_i, dst_i, rows_i, ev_i, nb):
      pltpu.sync_copy(src_hbm.at[pl.ds(base, nb)], src_i)
      pltpu.sync_copy(dst_hbm.at[pl.ds(base, nb)], dst_i)
      pltpu.async_copy(h_hbm.at[src_i], rows_i, sem).wait()
      pltpu.sync_copy(e_hbm.at[pl.ds(base, nb)], ev_i)

      def row(b, carry):
        for j in range(H // 16):
          sl = pl.ds(j * 16, 16)
          v = rows_i[b, sl] + ev_i[b, sl]
          rows_i[b, sl] = jnp.maximum(v, 0.0)
        return carry
      lax.fori_loop(0, nb, row, 0)

      pltpu.sync_copy(rows_i, acc_sh.at[dst_i], add=True)

    def chunk(i, carry):
      do_chunk(wid * EW + i * _CHUNK, src_v, dst_v, rows_v, ev_v, _CHUNK)
      return carry
    lax.fori_loop(0, n_full, chunk, 0)
    if rem:
      do_chunk(wid * EW + n_full * _CHUNK, src_r, dst_r, rows_r, ev_r, rem)

    plsc.subcore_barrier()
    pltpu.sync_copy(acc_sh.at[pl.ds(s * rows_per_tile, rows_per_tile)],
                    out_hbm.at[c, pl.ds(s * rows_per_tile, rows_per_tile)])

  mesh = plsc.VectorSubcoreMesh(core_axis_name="c", subcore_axis_name="s")
  scratch = [
      pltpu.VMEM_SHARED((N_pad, H), F32),
      pltpu.VMEM((_CHUNK,), jnp.int32),
      pltpu.VMEM((_CHUNK,), jnp.int32),
      pltpu.VMEM((_CHUNK, H), F32),
      pltpu.VMEM((_CHUNK, H), F32),
      pltpu.VMEM((max(rem, 8),), jnp.int32),
      pltpu.VMEM((max(rem, 8),), jnp.int32),
      pltpu.VMEM((max(rem, 8), H), F32),
      pltpu.VMEM((max(rem, 8), H), F32),
      pltpu.SemaphoreType.DMA,
  ]
  return pl.kernel(
      body,
      out_type=jax.ShapeDtypeStruct((_NC, N_pad, H), F32),
      mesh=mesh,
      scratch_types=scratch,
  ), N_pad


def kernel(x, edge_index, edge_attr, We, be, W1, b1, W2, b2):
  N, H = x.shape[0], We.shape[2]
  E = edge_attr.shape[0]
  src = edge_index[0].astype(jnp.int32)
  dst = edge_index[1].astype(jnp.int32)
  sc_edge_pass, N_pad = _make_sc_edge_pass(N, E, H)
  zeros = jnp.zeros((N_pad, H), dtype=F32)

  num_layers = We.shape[0]
  h = x
  for l in range(num_layers):
    e = _edge_proj(edge_attr, We[l], be[l])
    agg = sc_edge_pass(h, e, src, dst, zeros)
    h = _node_update(h, agg[0, :N], agg[1, :N], W1[l], b1[l], W2[l], b2[l],
                     relu_out=(l < num_layers - 1))
  return h
